# baseline (speedup 1.0000x reference)
import jax
import jax.numpy as jnp
from jax.experimental import pallas as pl
from jax.experimental.pallas import tpu as pltpu

B = 8
P = 784
D = 1024
K = 16384
Q = B * P
NN = 9

BQ = 896
BK = 2048


def _min_dist_kernel(qt_ref, m_ref, out_ref):
    nk = pl.num_programs(1)
    j = pl.program_id(1)
    qt = qt_ref[...]
    m = m_ref[...]
    qm = jnp.dot(m, qt, preferred_element_type=jnp.float32)
    m2 = jnp.sum(m * m, axis=1)
    part = m2[:, None] - 2.0 * qm
    bmin = jnp.min(part, axis=0)[None, None, :]

    @pl.when(j == 0)
    def _():
        out_ref[...] = bmin

    @pl.when(j > 0)
    def _():
        out_ref[...] = jnp.minimum(out_ref[...], bmin)

    @pl.when(j == nk - 1)
    def _():
        q2 = jnp.sum(qt * qt, axis=0)[None, None, :]
        out_ref[...] = jnp.sqrt(jnp.maximum(out_ref[...] + q2, 1e-12))


def _select_row_kernel(ps_ref, qt_ref, m_ref, d2_ref, sstar_ref, qsel_ref):
    j = pl.program_id(0)

    @pl.when(j == 0)
    def _():
        ps = ps_ref[...]
        sstar_ref[...] = jnp.max(ps, axis=1)[None, :]
        idx = jnp.argmax(ps, axis=1)
        flat = idx + jax.lax.iota(jnp.int32, B) * P
        onehot = (jax.lax.broadcasted_iota(jnp.int32, (Q, B), 0) ==
                  flat[None, :]).astype(jnp.float32)
        qsel_ref[...] = jnp.dot(qt_ref[...], onehot,
                                preferred_element_type=jnp.float32)

    qsel = qsel_ref[...]
    m = m_ref[...]
    qm = jnp.dot(m, qsel, preferred_element_type=jnp.float32)
    m2 = jnp.sum(m * m, axis=1)[:, None]
    q2 = jnp.sum(qsel * qsel, axis=0)[None, :]
    d2_ref[...] = q2 + m2 - 2.0 * qm


def _rescore_kernel(d2_ref, sstar_ref, out_ref):
    d = jnp.sqrt(jnp.maximum(d2_ref[...], 1e-12))
    col = jax.lax.broadcasted_iota(jnp.int32, (B, K), 1)
    nn = []
    for _ in range(NN):
        cur = jnp.min(d, axis=1)
        nn.append(cur)
        amin = jnp.argmin(d, axis=1)
        d = jnp.where(col == amin[:, None], jnp.inf, d)
    nn_dists = jnp.stack(nn, axis=1)
    sstar = sstar_ref[...][0, :]
    mx = nn_dists[:, NN - 1]
    weights = 1.0 - jnp.exp(sstar - mx) / jnp.sum(
        jnp.exp(nn_dists - mx[:, None]), axis=1)
    out_ref[...] = (weights * sstar)[None, :]


def kernel(queries, memory_bank):
    qt = queries.T

    patch_t = pl.pallas_call(
        _min_dist_kernel,
        grid=(Q // BQ, K // BK),
        in_specs=[
            pl.BlockSpec((D, BQ), lambda i, j: (0, i)),
            pl.BlockSpec((BK, D), lambda i, j: (j, 0)),
        ],
        out_specs=pl.BlockSpec((1, 1, BQ), lambda i, j: (i, 0, 0)),
        out_shape=jax.ShapeDtypeStruct((Q // BQ, 1, BQ), jnp.float32),
        compiler_params=pltpu.CompilerParams(
            dimension_semantics=("parallel", "arbitrary")),
    )(qt, memory_bank)
    patch_scores = patch_t.reshape(B, P)

    d2_cols, sstar = pl.pallas_call(
        _select_row_kernel,
        grid=(K // BK,),
        in_specs=[
            pl.BlockSpec((B, P), lambda j: (0, 0)),
            pl.BlockSpec((D, Q), lambda j: (0, 0)),
            pl.BlockSpec((BK, D), lambda j: (j, 0)),
        ],
        out_specs=[
            pl.BlockSpec((BK, B), lambda j: (j, 0)),
            pl.BlockSpec((1, B), lambda j: (0, 0)),
        ],
        out_shape=[
            jax.ShapeDtypeStruct((K, B), jnp.float32),
            jax.ShapeDtypeStruct((1, B), jnp.float32),
        ],
        scratch_shapes=[pltpu.VMEM((D, B), jnp.float32)],
        compiler_params=pltpu.CompilerParams(
            dimension_semantics=("arbitrary",)),
    )(patch_scores, qt, memory_bank)

    d2_rows = d2_cols.T

    image_scores = pl.pallas_call(
        _rescore_kernel,
        in_specs=[
            pl.BlockSpec((B, K), lambda: (0, 0)),
            pl.BlockSpec((1, B), lambda: (0, 0)),
        ],
        out_specs=pl.BlockSpec((1, B), lambda: (0, 0)),
        out_shape=jax.ShapeDtypeStruct((1, B), jnp.float32),
    )(d2_rows, sstar)[0, :]

    return image_scores, patch_scores


# speedup vs baseline: 1.2076x; 1.2076x over previous
import jax
import jax.numpy as jnp
from jax.experimental import pallas as pl
from jax.experimental.pallas import tpu as pltpu

B = 8
P = 784
D = 1024
K = 16384
Q = B * P
NN = 9

BQ = 896
BK = 2048
NQ = Q // BQ
NK = K // BK


def _min_dist_kernel(q_ref, m_ref, out_ref, mt_s, m2_s):
    j = pl.program_id(0)
    i = pl.program_id(1)

    @pl.when(i == 0)
    def _():
        m = m_ref[...]
        mt_s[...] = m.T
        m2_s[...] = jnp.sum(m * m, axis=1)[None, :]

    q = q_ref[...]
    qm = jnp.dot(q, mt_s[...], preferred_element_type=jnp.float32)
    part = m2_s[...] - 2.0 * qm
    bmin = jnp.min(part, axis=1)[:, None]
    row = pl.ds(i * BQ, BQ)

    @pl.when(j == 0)
    def _():
        out_ref[row, :] = bmin

    @pl.when(j > 0)
    def _():
        out_ref[row, :] = jnp.minimum(out_ref[row, :], bmin)

    @pl.when(j == NK - 1)
    def _():
        q2 = jnp.sum(q * q, axis=1)[:, None]
        out_ref[row, :] = jnp.sqrt(jnp.maximum(out_ref[row, :] + q2, 1e-12))


def _select_row_kernel(ps_ref, q_ref, m_ref, d2_ref, sstar_ref, qselt_s):
    j = pl.program_id(0)

    @pl.when(j == 0)
    def _():
        ps = ps_ref[...]
        sstar_ref[...] = jnp.max(ps, axis=1)[:, None]
        idx = jnp.argmax(ps, axis=1)
        flat = idx + jax.lax.iota(jnp.int32, B) * P
        onehot = (flat[:, None] ==
                  jax.lax.broadcasted_iota(jnp.int32, (B, Q), 1)).astype(jnp.float32)
        qsel = jnp.dot(onehot, q_ref[...],
                       preferred_element_type=jnp.float32)
        qselt_s[...] = qsel.T

    qselt = qselt_s[...]
    m = m_ref[...]
    qm = jnp.dot(m, qselt, preferred_element_type=jnp.float32)
    m2 = jnp.sum(m * m, axis=1)[:, None]
    q2 = jnp.sum(qselt * qselt, axis=0)[None, :]
    d2_ref[...] = q2 + m2 - 2.0 * qm


def _rescore_kernel(d2_ref, sstar_ref, out_ref):
    d = jnp.sqrt(jnp.maximum(d2_ref[...], 1e-12))
    col = jax.lax.broadcasted_iota(jnp.int32, (B, K), 1)
    nn = []
    for _ in range(NN):
        cur = jnp.min(d, axis=1)
        nn.append(cur)
        amin = jnp.argmin(d, axis=1)
        d = jnp.where(col == amin[:, None], jnp.inf, d)
    nn_dists = jnp.stack(nn, axis=1)
    sstar = sstar_ref[...][:, 0]
    mx = nn_dists[:, NN - 1]
    weights = 1.0 - jnp.exp(sstar - mx) / jnp.sum(
        jnp.exp(nn_dists - mx[:, None]), axis=1)
    out_ref[...] = (weights * sstar)[:, None]


def kernel(queries, memory_bank):
    patch_flat = pl.pallas_call(
        _min_dist_kernel,
        grid=(NK, NQ),
        in_specs=[
            pl.BlockSpec((BQ, D), lambda j, i: (i, 0)),
            pl.BlockSpec((BK, D), lambda j, i: (j, 0)),
        ],
        out_specs=pl.BlockSpec((Q, 1), lambda j, i: (0, 0)),
        out_shape=jax.ShapeDtypeStruct((Q, 1), jnp.float32),
        scratch_shapes=[
            pltpu.VMEM((D, BK), jnp.float32),
            pltpu.VMEM((1, BK), jnp.float32),
        ],
        compiler_params=pltpu.CompilerParams(
            dimension_semantics=("arbitrary", "arbitrary")),
    )(queries, memory_bank)
    patch_scores = patch_flat[:, 0].reshape(B, P)

    d2_cols, sstar = pl.pallas_call(
        _select_row_kernel,
        grid=(NK,),
        in_specs=[
            pl.BlockSpec((B, P), lambda j: (0, 0)),
            pl.BlockSpec((Q, D), lambda j: (0, 0)),
            pl.BlockSpec((BK, D), lambda j: (j, 0)),
        ],
        out_specs=[
            pl.BlockSpec((BK, B), lambda j: (j, 0)),
            pl.BlockSpec((B, 1), lambda j: (0, 0)),
        ],
        out_shape=[
            jax.ShapeDtypeStruct((K, B), jnp.float32),
            jax.ShapeDtypeStruct((B, 1), jnp.float32),
        ],
        scratch_shapes=[pltpu.VMEM((D, B), jnp.float32)],
        compiler_params=pltpu.CompilerParams(
            dimension_semantics=("arbitrary",)),
    )(patch_scores, queries, memory_bank)

    d2_rows = d2_cols.T

    image_scores = pl.pallas_call(
        _rescore_kernel,
        in_specs=[
            pl.BlockSpec((B, K), lambda: (0, 0)),
            pl.BlockSpec((B, 1), lambda: (0, 0)),
        ],
        out_specs=pl.BlockSpec((B, 1), lambda: (0, 0)),
        out_shape=jax.ShapeDtypeStruct((B, 1), jnp.float32),
    )(d2_rows, sstar)[:, 0]

    return image_scores, patch_scores


# speedup vs baseline: 1.2341x; 1.0219x over previous
import jax
import jax.numpy as jnp
from jax.experimental import pallas as pl
from jax.experimental.pallas import tpu as pltpu

B = 8
P = 784
D = 1024
K = 16384
Q = B * P
NN = 9

BQ = 896
BK = 2048
NQ = Q // BQ
NK = K // BK


def _min_dist_kernel(q_ref, m_ref, out_ref, m2_ref, mt_s):
    j = pl.program_id(0)
    i = pl.program_id(1)

    @pl.when(i == 0)
    def _():
        m = m_ref[...]
        mt_s[...] = -2.0 * m.T
        m2_ref[...] = jnp.sum(m * m, axis=1)[None, :]

    q = q_ref[...]
    qm = jnp.dot(q, mt_s[...], preferred_element_type=jnp.float32)
    part = m2_ref[...] + qm
    bmin = jnp.min(part, axis=1)[:, None]
    row = pl.ds(i * BQ, BQ)

    @pl.when(j == 0)
    def _():
        out_ref[row, :] = bmin

    @pl.when(j > 0)
    def _():
        out_ref[row, :] = jnp.minimum(out_ref[row, :], bmin)

    @pl.when(j == NK - 1)
    def _():
        q2 = jnp.sum(q * q, axis=1)[:, None]
        out_ref[row, :] = jnp.sqrt(jnp.maximum(out_ref[row, :] + q2, 1e-12))


def _select_score_kernel(ps_ref, q_ref, m_ref, m2_ref, out_ref,
                         qselt_s, d2t_s, sstar_s):
    j = pl.program_id(0)

    @pl.when(j == 0)
    def _():
        ps = ps_ref[...]
        sstar_s[...] = jnp.max(ps, axis=1)[:, None]
        idx = jnp.argmax(ps, axis=1)
        flat = idx + jax.lax.iota(jnp.int32, B) * P
        onehot = (flat[:, None] ==
                  jax.lax.broadcasted_iota(jnp.int32, (B, Q), 1)).astype(jnp.float32)
        qsel = jnp.dot(onehot, q_ref[...],
                       preferred_element_type=jnp.float32)
        qselt_s[...] = -2.0 * qsel.T

    qselt = qselt_s[...]
    m = m_ref[...]
    qm = jnp.dot(m, qselt, preferred_element_type=jnp.float32)
    q2 = 0.25 * jnp.sum(qselt * qselt, axis=0)[:, None]
    cols = pl.ds(j * BK, BK)
    d2t_s[:, cols] = qm.T + m2_ref[...] + q2

    @pl.when(j == NK - 1)
    def _():
        d = jnp.sqrt(jnp.maximum(d2t_s[...], 1e-12))
        col = jax.lax.broadcasted_iota(jnp.int32, (B, K), 1)
        nn = []
        for _ in range(NN):
            cur = jnp.min(d, axis=1)
            nn.append(cur)
            amin = jnp.argmin(d, axis=1)
            d = jnp.where(col == amin[:, None], jnp.inf, d)
        nn_dists = jnp.stack(nn, axis=1)
        sstar = sstar_s[...][:, 0]
        mx = nn_dists[:, NN - 1]
        weights = 1.0 - jnp.exp(sstar - mx) / jnp.sum(
            jnp.exp(nn_dists - mx[:, None]), axis=1)
        out_ref[...] = (weights * sstar)[:, None]


def kernel(queries, memory_bank):
    patch_flat, m2_all = pl.pallas_call(
        _min_dist_kernel,
        grid=(NK, NQ),
        in_specs=[
            pl.BlockSpec((BQ, D), lambda j, i: (i, 0)),
            pl.BlockSpec((BK, D), lambda j, i: (j, 0)),
        ],
        out_specs=[
            pl.BlockSpec((Q, 1), lambda j, i: (0, 0)),
            pl.BlockSpec((1, BK), lambda j, i: (0, j)),
        ],
        out_shape=[
            jax.ShapeDtypeStruct((Q, 1), jnp.float32),
            jax.ShapeDtypeStruct((1, K), jnp.float32),
        ],
        scratch_shapes=[
            pltpu.VMEM((D, BK), jnp.float32),
        ],
        compiler_params=pltpu.CompilerParams(
            dimension_semantics=("arbitrary", "arbitrary")),
    )(queries, memory_bank)
    patch_scores = patch_flat[:, 0].reshape(B, P)

    image_scores = pl.pallas_call(
        _select_score_kernel,
        grid=(NK,),
        in_specs=[
            pl.BlockSpec((B, P), lambda j: (0, 0)),
            pl.BlockSpec((Q, D), lambda j: (0, 0)),
            pl.BlockSpec((BK, D), lambda j: (j, 0)),
            pl.BlockSpec((1, BK), lambda j: (0, j)),
        ],
        out_specs=pl.BlockSpec((B, 1), lambda j: (0, 0)),
        out_shape=jax.ShapeDtypeStruct((B, 1), jnp.float32),
        scratch_shapes=[
            pltpu.VMEM((D, B), jnp.float32),
            pltpu.VMEM((B, K), jnp.float32),
            pltpu.VMEM((B, 1), jnp.float32),
        ],
        compiler_params=pltpu.CompilerParams(
            dimension_semantics=("arbitrary",)),
    )(patch_scores, queries, memory_bank, m2_all)[:, 0]

    return image_scores, patch_scores
